# baseline (speedup 1.0000x reference)
import math
from functools import partial

import jax
import jax.numpy as jnp
from jax.experimental import pallas as pl
from jax.experimental.pallas import tpu as pltpu


def _lrelu(x):
    return jnp.maximum(x, 0.01 * x)


def _raindrop_body(
    x_ref, times_ref, mask_ref,
    bn_g_ref, bn_b_ref, oe_ref, isa_ref, bidir_ref,
    projW_ref, projb_ref, qW_ref, qb_ref, kW_ref, kb_ref,
    sW_ref, sb_ref, ln_t_g_ref, ln_t_b_ref, eW_ref, eb_ref,
    ln_s_g_ref, ln_s_b_ref,
    out_ref, adj_ref,
    *, B, T, S, EMB, PE, ISA, KEEP,
):
    b = pl.program_id(0)

    x = x_ref[...]
    mean_t = jnp.mean(x, axis=(0, 2))
    var_t = jnp.mean(x * x, axis=(0, 2)) - mean_t * mean_t
    xb = x_ref[pl.ds(b, 1)][0]
    xn = (xb - mean_t[:, None]) / jnp.sqrt(var_t[:, None] + 1e-5)
    xn = xn * bn_g_ref[0][:, None] + bn_b_ref[0][:, None]

    oe = oe_ref[0]
    h = _lrelu(xn[:, :, None] * oe[None, :, :])
    msk = mask_ref[0]
    h = h * msk[:, :, None]

    trow = times_ref[0, 0]
    k2 = jax.lax.broadcasted_iota(jnp.int32, (1, PE // 2), 1).astype(jnp.float32) * 2.0
    div = jnp.exp(k2 * (-math.log(10000.0) / PE))
    ang = trow[:, None] * div
    pe = jnp.concatenate([jnp.sin(ang), jnp.cos(ang)], axis=-1)

    projW = projW_ref[...]
    projb = projb_ref[0]

    adj = None
    for lay in range(2):
        isa_l = isa_ref[lay, 0]
        wb = bidir_ref[lay, 0]
        bid = _lrelu(jnp.dot(wb, wb.T, preferred_element_type=jnp.float32))

        hp = jnp.dot(h.reshape(T * S, EMB), projW,
                     preferred_element_type=jnp.float32) + projb
        hp_isa = hp[:, :ISA]
        hp_pe = hp[:, ISA:].reshape(T, S, PE)
        term1 = jnp.dot(hp_isa, isa_l.T,
                        preferred_element_type=jnp.float32).reshape(T, S, S)
        c = jnp.sum(hp_pe * pe[:, None, :], axis=-1)
        alpha = _lrelu(term1 + c[:, :, None])

        if lay == 0:
            fin = _lrelu(bid[None] * alpha)
        else:
            msum = jnp.sum(msk, axis=0)
            asum = jnp.sum(alpha, axis=0)
            adj_pre = _lrelu(asum / msum[:, None])
            lo0 = jnp.min(adj_pre) - 1.0
            hi0 = jnp.max(adj_pre) + 1.0

            def _bisect(_, carry):
                lo, hi = carry
                mid = 0.5 * (lo + hi)
                c = jnp.sum(jnp.where(adj_pre > mid, 1.0, 0.0))
                pred = c >= KEEP
                return jnp.where(pred, mid, lo), jnp.where(pred, hi, mid)

            lo, hi = jax.lax.fori_loop(0, 50, _bisect, (lo0, hi0))
            adj = adj_pre * jnp.where(adj_pre > lo, 1.0, 0.0)
            adj_ref[0, 0] = adj
            fin = _lrelu(bid[None] * alpha * adj[None])

        fcat = jnp.concatenate(
            [jnp.maximum(fin, 0.0), jnp.minimum(fin, 0.0)], axis=2)
        hcat = jnp.concatenate(
            [_lrelu(h), jnp.minimum(h, 0.01 * h)], axis=1)
        hnew = jax.lax.dot_general(
            fcat, hcat, (((2,), (1,)), ((0,), (0,))),
            preferred_element_type=jnp.float32)
        h = _lrelu(hnew)

    ht = jnp.transpose(h, (1, 0, 2))
    pe_b = jnp.broadcast_to(pe[None, :, :], (S, T, PE))
    Hc = jnp.concatenate([ht, pe_b], axis=-1)
    Hc2 = Hc.reshape(S * T, EMB + PE)
    Q = (jnp.dot(Hc2, qW_ref[...], preferred_element_type=jnp.float32)
         + qb_ref[0]).reshape(S, T, -1)
    K = (jnp.dot(Hc2, kW_ref[...], preferred_element_type=jnp.float32)
         + kb_ref[0]).reshape(S, T, -1)
    kv = jnp.sum(K * sW_ref[...][None, :, :], axis=1)
    beta = jnp.sum(Q * kv[:, None, :], axis=-1) + sb_ref[0, 0]

    m = jnp.mean(beta)
    v = jnp.mean((beta - m) * (beta - m))
    beta = (beta - m) / jnp.sqrt(v + 1e-5) * ln_t_g_ref[...] + ln_t_b_ref[...]

    ctx = jnp.sum(beta[:, :, None] * Hc, axis=1)
    o = _lrelu(ctx)
    o = _lrelu(jnp.dot(o, eW_ref[...], preferred_element_type=jnp.float32)
               + eb_ref[0])
    m2 = jnp.mean(o)
    v2 = jnp.mean((o - m2) * (o - m2))
    o = (o - m2) / jnp.sqrt(v2 + 1e-5) * ln_s_g_ref[...] + ln_s_b_ref[...]
    out_ref[0, 0] = o


def _sim_body(adj_ref, sim_ref, *, B, S):
    adj = adj_ref[...]
    sumsq = jnp.sum(adj * adj)
    sb = jnp.sum(adj, axis=0)
    norm = jnp.sum(sb * sb)
    val = (B * sumsq - norm) / ((B - 1) ** 2 * S * S)
    sim_ref[...] = jnp.broadcast_to(val, (1, 1))


def kernel(x, times, mask, params):
    p = params
    B, T, S, _ = x.shape
    H = p["isa"].shape[1]
    ISA = p["isa"].shape[-1]
    EMB = p["projW"].shape[0]
    PE = p["projW"].shape[1] - ISA
    OH = p["eW"].shape[1]
    KEEP = (S * S) - math.floor(S * S * 0.5)

    x3 = x[..., 0]
    oe = p["obs_emb"].reshape(S, H, EMB).transpose(1, 0, 2)
    times3 = times.reshape(B, 1, T)

    f32 = jnp.float32
    row = lambda a: a.reshape(1, -1)

    full = lambda shape: pl.BlockSpec(shape, lambda b, h: tuple(0 for _ in shape))
    per_b = lambda shape: pl.BlockSpec(shape, lambda b, h: (b,) + tuple(0 for _ in shape[1:]))

    in_specs = [
        full((B, T, S)),
        per_b((1, 1, T)),
        per_b((1, T, S)),
        full((1, T)), full((1, T)),
        pl.BlockSpec((1, S, EMB), lambda b, h: (h, 0, 0)),
        pl.BlockSpec((2, 1, S, ISA), lambda b, h: (0, h, 0, 0)),
        pl.BlockSpec((2, 1, S, ISA), lambda b, h: (0, h, 0, 0)),
        full((EMB, ISA + PE)), full((1, ISA + PE)),
        full((EMB + PE, p["qW"].shape[1])), full((1, p["qW"].shape[1])),
        full((EMB + PE, p["kW"].shape[1])), full((1, p["kW"].shape[1])),
        full((T, 1)), full((1, 1)),
        full((S, T)), full((S, T)),
        full((EMB + PE, OH)), full((1, OH)),
        full((S, OH)), full((S, OH)),
    ]
    out_specs = [
        pl.BlockSpec((1, 1, S, OH), lambda b, h: (b, h, 0, 0)),
        pl.BlockSpec((1, 1, S, S), lambda b, h: (b, h, 0, 0)),
    ]
    out_shape = [
        jax.ShapeDtypeStruct((B, H, S, OH), f32),
        jax.ShapeDtypeStruct((B, H, S, S), f32),
    ]

    body = partial(_raindrop_body, B=B, T=T, S=S, EMB=EMB, PE=PE, ISA=ISA,
                   KEEP=KEEP)
    outk, adj = pl.pallas_call(
        body,
        grid=(B, H),
        in_specs=in_specs,
        out_specs=out_specs,
        out_shape=out_shape,
        compiler_params=pltpu.CompilerParams(
            dimension_semantics=("parallel", "parallel")),
    )(
        x3, times3, mask,
        row(p["bn_g"]), row(p["bn_b"]), oe, p["isa"], p["bidir"],
        p["projW"], row(p["projb"]), p["qW"], row(p["qb"]),
        p["kW"], row(p["kb"]), p["sW"], row(p["sb"]),
        p["ln_t_g"], p["ln_t_b"], p["eW"], row(p["eb"]),
        p["ln_s_g"], p["ln_s_b"],
    )

    sim = pl.pallas_call(
        partial(_sim_body, B=B, S=S),
        out_shape=jax.ShapeDtypeStruct((1, 1), f32),
    )(adj)[0, 0]

    out = outk.transpose(0, 2, 1, 3).reshape(B, S, H * OH)
    return out, sim


# speedup vs baseline: 1.6210x; 1.6210x over previous
import math
from functools import partial

import jax
import jax.numpy as jnp
from jax.experimental import pallas as pl
from jax.experimental.pallas import tpu as pltpu


def _lrelu(x):
    return jnp.maximum(x, 0.01 * x)


def _raindrop_body(
    x_ref, times_ref, mask_ref,
    bn_g_ref, bn_b_ref, oe_ref, isa_ref, bidir_ref,
    projW_ref, projb_ref, qW_ref, qb_ref, kW_ref, kb_ref,
    sW_ref, sb_ref, ln_t_g_ref, ln_t_b_ref, eW_ref, eb_ref,
    ln_s_g_ref, ln_s_b_ref,
    out_ref, adj_ref,
    *, B, T, S, EMB, PE, ISA, KEEP,
):
    x = x_ref[...]
    mean_t = jnp.mean(x, axis=(0, 2), keepdims=True)
    var_t = jnp.mean(x * x, axis=(0, 2), keepdims=True) - mean_t * mean_t
    xn = (x - mean_t) / jnp.sqrt(var_t + 1e-5)
    xn = xn * bn_g_ref[0][None, :, None] + bn_b_ref[0][None, :, None]

    oe = oe_ref[0]
    h = _lrelu(xn[..., None] * oe[None, None, :, :])
    msk = mask_ref[...]
    h = h * msk[..., None]

    tms = times_ref[...]
    k2 = jax.lax.broadcasted_iota(jnp.int32, (1, PE // 2), 1).astype(jnp.float32) * 2.0
    div = jnp.exp(k2 * (-math.log(10000.0) / PE))
    ang = tms[:, :, None] * div[None, :, :]
    pe = jnp.concatenate([jnp.sin(ang), jnp.cos(ang)], axis=-1)

    projW = projW_ref[...]
    projb = projb_ref[0]

    adj = None
    for lay in range(2):
        isa_l = isa_ref[lay, 0]
        wb = bidir_ref[lay, 0]
        bid = _lrelu(jnp.dot(wb, wb.T, preferred_element_type=jnp.float32))

        hp = jnp.dot(h.reshape(B * T * S, EMB), projW,
                     preferred_element_type=jnp.float32) + projb
        hp_isa = hp[:, :ISA]
        hp_pe = hp[:, ISA:].reshape(B, T, S, PE)
        term1 = jnp.dot(hp_isa, isa_l.T,
                        preferred_element_type=jnp.float32).reshape(B, T, S, S)
        c = jnp.sum(hp_pe * pe[:, :, None, :], axis=-1)
        alpha = _lrelu(term1 + c[..., None])

        if lay == 0:
            fin = _lrelu(bid[None, None] * alpha)
        else:
            msum = jnp.sum(msk, axis=1)
            asum = jnp.sum(alpha, axis=1)
            adj_pre = _lrelu(asum / msum[:, :, None])
            amin = jnp.min(adj_pre, axis=(1, 2))[:, None] - 1.0
            amax = jnp.max(adj_pre, axis=(1, 2))[:, None] + 1.0

            def _bisect(_, carry):
                lo, hi = carry
                mid = 0.5 * (lo + hi)
                cnt = jnp.sum(
                    jnp.where(adj_pre > mid[:, :, None], 1.0, 0.0),
                    axis=(1, 2))[:, None]
                pred = cnt >= KEEP
                return (jnp.where(pred, mid, lo), jnp.where(pred, hi, mid))

            lo, hi = jax.lax.fori_loop(0, 50, _bisect, (amin, amax))
            adj = adj_pre * jnp.where(adj_pre > lo[:, :, None], 1.0, 0.0)
            adj_ref[:, 0] = adj
            fin = _lrelu(bid[None, None] * alpha * adj[:, None, :, :])

        fcat = jnp.concatenate(
            [jnp.maximum(fin, 0.0), jnp.minimum(fin, 0.0)],
            axis=3).reshape(B * T, S, 2 * S)
        hbt = h.reshape(B * T, S, EMB)
        hcat = jnp.concatenate(
            [_lrelu(hbt), jnp.minimum(hbt, 0.01 * hbt)], axis=1)
        hnew = jax.lax.dot_general(
            fcat, hcat, (((2,), (1,)), ((0,), (0,))),
            preferred_element_type=jnp.float32)
        h = _lrelu(hnew).reshape(B, T, S, EMB)

    pe_b = jnp.broadcast_to(pe[:, :, None, :], (B, T, S, PE))
    Hc = jnp.concatenate([h, pe_b], axis=-1)
    Hc2 = Hc.reshape(B * T * S, EMB + PE)
    Q = (jnp.dot(Hc2, qW_ref[...], preferred_element_type=jnp.float32)
         + qb_ref[0]).reshape(B, T, S, -1)
    K = (jnp.dot(Hc2, kW_ref[...], preferred_element_type=jnp.float32)
         + kb_ref[0]).reshape(B, T, S, -1)
    sw = sW_ref[...][None, :, None, 0, None]
    kv = jnp.sum(K * sw, axis=1)
    beta = jnp.sum(Q * kv[:, None, :, :], axis=-1) + sb_ref[0, 0]

    m = jnp.mean(beta, axis=(1, 2), keepdims=True)
    v = jnp.mean((beta - m) * (beta - m), axis=(1, 2), keepdims=True)
    beta = (beta - m) / jnp.sqrt(v + 1e-5) * ln_t_g_ref[...][None] \
        + ln_t_b_ref[...][None]

    ctx = jnp.sum(beta[..., None] * Hc, axis=1)
    o = _lrelu(ctx)
    o = _lrelu(jnp.dot(o.reshape(B * S, EMB + PE), eW_ref[...],
                       preferred_element_type=jnp.float32)
               + eb_ref[0]).reshape(B, S, -1)
    m2 = jnp.mean(o, axis=(1, 2), keepdims=True)
    v2 = jnp.mean((o - m2) * (o - m2), axis=(1, 2), keepdims=True)
    o = (o - m2) / jnp.sqrt(v2 + 1e-5) * ln_s_g_ref[...][None] \
        + ln_s_b_ref[...][None]
    out_ref[:, 0] = o


def _sim_body(adj_ref, sim_ref, *, B, S):
    adj = adj_ref[...]
    sumsq = jnp.sum(adj * adj)
    sb = jnp.sum(adj, axis=0)
    norm = jnp.sum(sb * sb)
    val = (B * sumsq - norm) / ((B - 1) ** 2 * S * S)
    sim_ref[...] = jnp.broadcast_to(val, (1, 1))


def kernel(x, times, mask, params):
    p = params
    B, T, S, _ = x.shape
    H = p["isa"].shape[1]
    ISA = p["isa"].shape[-1]
    EMB = p["projW"].shape[0]
    PE = p["projW"].shape[1] - ISA
    OH = p["eW"].shape[1]
    KEEP = (S * S) - math.floor(S * S * 0.5)

    x3 = x[..., 0]
    oe = p["obs_emb"].reshape(S, H, EMB).transpose(1, 0, 2)

    f32 = jnp.float32
    row = lambda a: a.reshape(1, -1)

    full = lambda shape: pl.BlockSpec(shape, lambda hh: tuple(0 for _ in shape))

    in_specs = [
        full((B, T, S)),
        full((B, T)),
        full((B, T, S)),
        full((1, T)), full((1, T)),
        pl.BlockSpec((1, S, EMB), lambda hh: (hh, 0, 0)),
        pl.BlockSpec((2, 1, S, ISA), lambda hh: (0, hh, 0, 0)),
        pl.BlockSpec((2, 1, S, ISA), lambda hh: (0, hh, 0, 0)),
        full((EMB, ISA + PE)), full((1, ISA + PE)),
        full((EMB + PE, p["qW"].shape[1])), full((1, p["qW"].shape[1])),
        full((EMB + PE, p["kW"].shape[1])), full((1, p["kW"].shape[1])),
        full((T, 1)), full((1, 1)),
        full((T, S)), full((T, S)),
        full((EMB + PE, OH)), full((1, OH)),
        full((S, OH)), full((S, OH)),
    ]
    out_specs = [
        pl.BlockSpec((B, 1, S, OH), lambda hh: (0, hh, 0, 0)),
        pl.BlockSpec((B, 1, S, S), lambda hh: (0, hh, 0, 0)),
    ]
    out_shape = [
        jax.ShapeDtypeStruct((B, H, S, OH), f32),
        jax.ShapeDtypeStruct((B, H, S, S), f32),
    ]

    body = partial(_raindrop_body, B=B, T=T, S=S, EMB=EMB, PE=PE, ISA=ISA,
                   KEEP=KEEP)
    outk, adj = pl.pallas_call(
        body,
        grid=(H,),
        in_specs=in_specs,
        out_specs=out_specs,
        out_shape=out_shape,
        compiler_params=pltpu.CompilerParams(
            dimension_semantics=("parallel",)),
    )(
        x3, times, mask,
        row(p["bn_g"]), row(p["bn_b"]), oe, p["isa"], p["bidir"],
        p["projW"], row(p["projb"]), p["qW"], row(p["qb"]),
        p["kW"], row(p["kb"]), p["sW"], row(p["sb"]),
        p["ln_t_g"].T, p["ln_t_b"].T, p["eW"], row(p["eb"]),
        p["ln_s_g"], p["ln_s_b"],
    )

    sim = pl.pallas_call(
        partial(_sim_body, B=B, S=S),
        out_shape=jax.ShapeDtypeStruct((1, 1), f32),
    )(adj)[0, 0]

    out = outk.transpose(0, 2, 1, 3).reshape(B, S, H * OH)
    return out, sim
